# Initial kernel scaffold; baseline (speedup 1.0000x reference)
import jax
import jax.numpy as jnp
from jax.experimental import pallas as pl


def kernel(x, edge_index, W1, b1, W2, b2):
    raise NotImplementedError("write your pallas kernel here")


# speedup vs baseline: 28.5477x; 28.5477x over previous
import functools

import jax
import jax.numpy as jnp
from jax import lax
from jax.experimental import pallas as pl
from jax.experimental.pallas import tpu as pltpu
from jax.experimental.pallas import tpu_sc as plsc

NC = 2
NS = 16
NW = NC * NS
LANE = 128
KSUB = 2
STAGE_BYTES = 65536


def _cdiv(a, b):
    return (a + b - 1) // b


def _pick_sub(rows_per_tile, feat):
    best = 8
    for sub in range(8, rows_per_tile + 1, 8):
        if rows_per_tile % sub == 0 and sub * feat * 4 <= STAGE_BYTES:
            best = sub
    return best


def _make_deg_kernel(npad, rpad):
    rpw = rpad // NW
    rows_per_tile = npad // NS
    sub = _pick_sub(rows_per_tile, 16)
    pieces = rows_per_tile // sub
    mesh = plsc.VectorSubcoreMesh(core_axis_name="c", subcore_axis_name="s",
                                  num_cores=NC, num_subcores=NS)

    @functools.partial(
        pl.kernel, mesh=mesh,
        compiler_params=pltpu.CompilerParams(use_tc_tiling_on_sc=False),
        out_type=jax.ShapeDtypeStruct((NC, npad, 16), jnp.float32),
        scratch_types=[
            pltpu.VMEM((rpw, LANE), jnp.int32),
            pltpu.VMEM((LANE, 16), jnp.float32),
            pltpu.VMEM((sub, 16), jnp.float32),
            pltpu.VMEM_SHARED((npad, 16), jnp.float32),
        ],
    )
    def deg_kernel(dst_hbm, ones_hbm, zrows_hbm, out_hbm, idx_v, ones_v,
                   stage_v, acc_sh):
        cid = lax.axis_index("c")
        sid = lax.axis_index("s")
        wid = sid * NC + cid
        r0 = sid * rows_per_tile
        for pi in range(rpw // 8):
            sl = pl.ds(pi * 8, 8)
            pltpu.sync_copy(dst_hbm.at[wid, sl], idx_v.at[sl])
        pltpu.sync_copy(ones_hbm, ones_v)
        pltpu.sync_copy(zrows_hbm, stage_v)
        for p in range(pieces):
            pltpu.sync_copy(stage_v, acc_sh.at[pl.ds(r0 + p * sub, sub)])
        plsc.subcore_barrier()

        def body(i, carry):
            for j in range(KSUB):
                pltpu.sync_copy(ones_v, acc_sh.at[idx_v.at[i * KSUB + j]],
                                add=True)
            return carry

        lax.fori_loop(0, rpw // KSUB, body, 0)
        plsc.subcore_barrier()
        for p in range(pieces):
            sl = pl.ds(r0 + p * sub, sub)
            pltpu.sync_copy(acc_sh.at[sl], stage_v)
            pltpu.sync_copy(stage_v, out_hbm.at[cid, sl])

    return deg_kernel


def _make_agg_kernel(npad, rpad, feat, feature_split):
    nworkers = NS if feature_split else NW
    rpw = rpad // nworkers
    nsub = rpw // KSUB
    rows_per_tile = npad // NS
    sub = _pick_sub(rows_per_tile, feat)
    pieces = rows_per_tile // sub
    mesh = plsc.VectorSubcoreMesh(core_axis_name="c", subcore_axis_name="s",
                                  num_cores=NC, num_subcores=NS)

    @functools.partial(
        pl.kernel, mesh=mesh,
        compiler_params=pltpu.CompilerParams(use_tc_tiling_on_sc=False),
        out_type=jax.ShapeDtypeStruct((NC, npad, feat), jnp.float32),
        scratch_types=[
            pltpu.VMEM((rpw, LANE), jnp.int32),
            pltpu.VMEM((rpw, LANE), jnp.int32),
            [pltpu.VMEM((KSUB, LANE, feat), jnp.float32)] * 2,
            pltpu.VMEM((sub, feat), jnp.float32),
            pltpu.VMEM_SHARED((npad, feat), jnp.float32),
            [pltpu.SemaphoreType.DMA] * 2,
        ],
    )
    def agg_kernel(tab0_hbm, tab1_hbm, src_hbm, dst_hbm, zrows_hbm, out_hbm,
                   src_v, dst_v, rows_v, stage_v, acc_sh, sem):
        cid = lax.axis_index("c")
        sid = lax.axis_index("s")
        wid = sid if feature_split else sid * NC + cid
        r0 = sid * rows_per_tile
        pc = 8
        for pi in range(rpw // pc):
            sl = pl.ds(pi * pc, pc)
            pltpu.async_copy(src_hbm.at[wid, sl], src_v.at[sl], sem[0])
            pltpu.async_copy(dst_hbm.at[wid, sl], dst_v.at[sl], sem[1])
        for pi in range(rpw // pc):
            sl = pl.ds(pi * pc, pc)
            pltpu.make_async_copy(src_hbm.at[wid, sl], src_v.at[sl],
                                  sem[0]).wait()
            pltpu.make_async_copy(dst_hbm.at[wid, sl], dst_v.at[sl],
                                  sem[1]).wait()
        pltpu.sync_copy(zrows_hbm, stage_v)
        for p in range(pieces):
            pltpu.sync_copy(stage_v, acc_sh.at[pl.ds(r0 + p * sub, sub)])
        plsc.subcore_barrier()

        def edge_loop(table_hbm):
            def fire(b, t):
                for j in range(KSUB):
                    pltpu.async_copy(table_hbm.at[src_v.at[t * KSUB + j]],
                                     rows_v[b].at[j], sem[b])

            def drain_scatter(b, t):
                for j in range(KSUB):
                    pltpu.make_async_copy(table_hbm.at[src_v.at[t * KSUB + j]],
                                          rows_v[b].at[j], sem[b]).wait()
                for j in range(KSUB):
                    pltpu.sync_copy(rows_v[b].at[j],
                                    acc_sh.at[dst_v.at[t * KSUB + j]],
                                    add=True)

            fire(0, 0)
            fire(1, 1)

            def body(i, carry):
                for b in range(2):
                    t = 2 * i + b
                    drain_scatter(b, t)
                    fire(b, t + 2)
                return carry

            lax.fori_loop(0, nsub // 2 - 1, body, 0)
            drain_scatter(0, nsub - 2)
            drain_scatter(1, nsub - 1)

        if feature_split:
            @pl.when(cid == 0)
            def _():
                edge_loop(tab0_hbm)

            @pl.when(cid == 1)
            def _():
                edge_loop(tab1_hbm)
        else:
            edge_loop(tab0_hbm)

        plsc.subcore_barrier()
        for p in range(pieces):
            sl = pl.ds(r0 + p * sub, sub)
            pltpu.sync_copy(acc_sh.at[sl], stage_v)
            pltpu.sync_copy(stage_v, out_hbm.at[cid, sl])

    return agg_kernel


def _tc1(x, w1, degp, bn):
    n, in_ch = x.shape
    hid = w1.shape[1]
    half = hid // 2

    def body(x_ref, w_ref, d_ref, lo_ref, hi_ref, dis_ref):
        deg = d_ref[0] + d_ref[1] + 1.0
        dis16 = lax.rsqrt(deg)
        dis_ref[...] = dis16
        h = jnp.dot(x_ref[...], w_ref[...], preferred_element_type=jnp.float32)
        hs = h * dis16[:, 0:1]
        lo_ref[...] = hs[:, :half]
        hi_ref[...] = hs[:, half:]

    return pl.pallas_call(
        body,
        grid=(n // bn,),
        in_specs=[
            pl.BlockSpec((bn, in_ch), lambda i: (i, 0)),
            pl.BlockSpec((in_ch, hid), lambda i: (0, 0)),
            pl.BlockSpec((NC, bn, 16), lambda i: (0, i, 0)),
        ],
        out_specs=[
            pl.BlockSpec((bn, half), lambda i: (i, 0)),
            pl.BlockSpec((bn, half), lambda i: (i, 0)),
            pl.BlockSpec((bn, 16), lambda i: (i, 0)),
        ],
        out_shape=[
            jax.ShapeDtypeStruct((n, half), jnp.float32),
            jax.ShapeDtypeStruct((n, half), jnp.float32),
            jax.ShapeDtypeStruct((n, 16), jnp.float32),
        ],
    )(x, w1, degp)


def _tc2(p, hs_lo, hs_hi, dis16, b1, w2, bn):
    n, half = hs_lo.shape
    dim = w2.shape[1]

    qh = dim // 2

    def body(p_ref, lo_ref, hi_ref, dis_ref, b_ref, w_ref, o_lo_ref,
             o_hi_ref):
        dis = dis_ref[:, 0:1]
        agg = jnp.concatenate(
            [p_ref[0] + lo_ref[...], p_ref[1] + hi_ref[...]], axis=1)
        z = jnp.maximum(agg * dis + b_ref[...], 0.0)
        hs2 = jnp.dot(z, w_ref[...], preferred_element_type=jnp.float32) * dis
        o_lo_ref[...] = hs2[:, :qh]
        o_hi_ref[...] = hs2[:, qh:]

    return pl.pallas_call(
        body,
        grid=(n // bn,),
        in_specs=[
            pl.BlockSpec((NC, bn, half), lambda i: (0, i, 0)),
            pl.BlockSpec((bn, half), lambda i: (i, 0)),
            pl.BlockSpec((bn, half), lambda i: (i, 0)),
            pl.BlockSpec((bn, 16), lambda i: (i, 0)),
            pl.BlockSpec((1, 2 * half), lambda i: (0, 0)),
            pl.BlockSpec((2 * half, dim), lambda i: (0, 0)),
        ],
        out_specs=[
            pl.BlockSpec((bn, qh), lambda i: (i, 0)),
            pl.BlockSpec((bn, qh), lambda i: (i, 0)),
        ],
        out_shape=[
            jax.ShapeDtypeStruct((n, qh), jnp.float32),
            jax.ShapeDtypeStruct((n, qh), jnp.float32),
        ],
    )(p, hs_lo, hs_hi, dis16, b1, w2)


def _tc3(q, hs2_lo, hs2_hi, dis16, b2, bn):
    n, qh = hs2_lo.shape
    dim = 2 * qh

    def body(q_ref, lo_ref, hi_ref, dis_ref, b_ref, o_ref):
        agg = jnp.concatenate(
            [q_ref[0] + lo_ref[...], q_ref[1] + hi_ref[...]], axis=1)
        o_ref[...] = agg * dis_ref[:, 0:1] + b_ref[...]

    return pl.pallas_call(
        body,
        grid=(n // bn,),
        in_specs=[
            pl.BlockSpec((NC, bn, qh), lambda i: (0, i, 0)),
            pl.BlockSpec((bn, qh), lambda i: (i, 0)),
            pl.BlockSpec((bn, qh), lambda i: (i, 0)),
            pl.BlockSpec((bn, 16), lambda i: (i, 0)),
            pl.BlockSpec((1, dim), lambda i: (0, 0)),
        ],
        out_specs=pl.BlockSpec((bn, dim), lambda i: (i, 0)),
        out_shape=jax.ShapeDtypeStruct((n, dim), jnp.float32),
    )(q, hs2_lo, hs2_hi, dis16, b2)


def kernel(x, edge_index, W1, b1, W2, b2):
    n, _ = x.shape
    hid = W1.shape[1]
    dim = W2.shape[1]
    e = edge_index.shape[1]
    assert e % LANE == 0
    r = e // LANE
    npad = _cdiv(n, NS * 8) * NS * 8
    rpad = _cdiv(r, 8 * NW) * 8 * NW
    bn = 1000 if n % 1000 == 0 else 8
    half = hid // 2

    src, dst = edge_index[0], edge_index[1]
    npadrows = rpad - r
    pad_e = npadrows * LANE
    pad_src = (jnp.arange(pad_e, dtype=jnp.int32) % n).reshape(npadrows, LANE)
    pad_dst = (n + jnp.arange(pad_e, dtype=jnp.int32) % (npad - n)).reshape(
        npadrows, LANE)
    srcp = jnp.concatenate([src.reshape(r, LANE), pad_src], axis=0)
    dstp = jnp.concatenate([dst.reshape(r, LANE), pad_dst], axis=0)
    ones16 = jnp.ones((LANE, 16), jnp.float32)

    rows_per_tile = npad // NS

    def zrows(feat):
        return jnp.zeros((_pick_sub(rows_per_tile, feat), feat), jnp.float32)

    srcp16 = srcp.reshape(NS, rpad // NS, LANE)
    dstp16 = dstp.reshape(NS, rpad // NS, LANE)
    srcp32 = srcp.reshape(NW, rpad // NW, LANE)
    dstp32 = dstp.reshape(NW, rpad // NW, LANE)
    degp = _make_deg_kernel(npad, rpad)(dstp32, ones16, zrows(16))
    hs_lo, hs_hi, dis16 = _tc1(x, W1, degp, bn)
    p = _make_agg_kernel(npad, rpad, half, feature_split=True)(
        hs_lo, hs_hi, srcp16, dstp16, zrows(half))
    hs2_lo, hs2_hi = _tc2(p, hs_lo, hs_hi, dis16, b1.reshape(1, hid), W2, bn)
    q = _make_agg_kernel(npad, rpad, dim // 2, feature_split=True)(
        hs2_lo, hs2_hi, srcp16, dstp16, zrows(dim // 2))
    return _tc3(q, hs2_lo, hs2_hi, dis16, b2.reshape(1, dim), bn)


# speedup vs baseline: 28.6415x; 1.0033x over previous
import functools

import jax
import jax.numpy as jnp
from jax import lax
from jax.experimental import pallas as pl
from jax.experimental.pallas import tpu as pltpu
from jax.experimental.pallas import tpu_sc as plsc

NC = 2
NS = 16
NW = NC * NS
LANE = 128
KSUB = 2
STAGE_BYTES = 65536


def _cdiv(a, b):
    return (a + b - 1) // b


def _pick_sub(rows_per_tile, feat):
    best = 8
    for sub in range(8, rows_per_tile + 1, 8):
        if rows_per_tile % sub == 0 and sub * feat * 4 <= STAGE_BYTES:
            best = sub
    return best


def _make_deg_kernel(npad, rpad):
    rpw = rpad // NW
    rows_per_tile = npad // NS
    sub = _pick_sub(rows_per_tile, 16)
    pieces = rows_per_tile // sub
    mesh = plsc.VectorSubcoreMesh(core_axis_name="c", subcore_axis_name="s",
                                  num_cores=NC, num_subcores=NS)

    @functools.partial(
        pl.kernel, mesh=mesh,
        compiler_params=pltpu.CompilerParams(use_tc_tiling_on_sc=False),
        out_type=jax.ShapeDtypeStruct((NC, npad, 16), jnp.float32),
        scratch_types=[
            pltpu.VMEM((rpw, LANE), jnp.int32),
            pltpu.VMEM((LANE, 16), jnp.float32),
            pltpu.VMEM((sub, 16), jnp.float32),
            pltpu.VMEM_SHARED((npad, 16), jnp.float32),
        ],
    )
    def deg_kernel(dst_hbm, ones_hbm, zrows_hbm, out_hbm, idx_v, ones_v,
                   stage_v, acc_sh):
        cid = lax.axis_index("c")
        sid = lax.axis_index("s")
        wid = sid * NC + cid
        r0 = sid * rows_per_tile
        for pi in range(rpw // 8):
            sl = pl.ds(pi * 8, 8)
            pltpu.sync_copy(dst_hbm.at[wid, sl], idx_v.at[sl])
        pltpu.sync_copy(ones_hbm, ones_v)
        pltpu.sync_copy(zrows_hbm, stage_v)
        for p in range(pieces):
            pltpu.sync_copy(stage_v, acc_sh.at[pl.ds(r0 + p * sub, sub)])
        plsc.subcore_barrier()

        def body(i, carry):
            for j in range(KSUB):
                pltpu.sync_copy(ones_v, acc_sh.at[idx_v.at[i * KSUB + j]],
                                add=True)
            return carry

        lax.fori_loop(0, rpw // KSUB, body, 0)
        plsc.subcore_barrier()
        for p in range(pieces):
            sl = pl.ds(r0 + p * sub, sub)
            pltpu.sync_copy(acc_sh.at[sl], stage_v)
            pltpu.sync_copy(stage_v, out_hbm.at[cid, sl])

    return deg_kernel


def _make_agg_kernel(npad, rpad, feat, feature_split):
    nworkers = NS if feature_split else NW
    rpw = rpad // nworkers
    nsub = rpw // KSUB
    rows_per_tile = npad // NS
    sub = _pick_sub(rows_per_tile, feat)
    pieces = rows_per_tile // sub
    mesh = plsc.VectorSubcoreMesh(core_axis_name="c", subcore_axis_name="s",
                                  num_cores=NC, num_subcores=NS)

    @functools.partial(
        pl.kernel, mesh=mesh,
        compiler_params=pltpu.CompilerParams(use_tc_tiling_on_sc=False),
        out_type=jax.ShapeDtypeStruct((NC, npad, feat), jnp.float32),
        scratch_types=[
            pltpu.VMEM((rpw, LANE), jnp.int32),
            pltpu.VMEM((rpw, LANE), jnp.int32),
            [pltpu.VMEM((KSUB, LANE, feat), jnp.float32)] * 2,
            pltpu.VMEM((sub, feat), jnp.float32),
            pltpu.VMEM_SHARED((npad, feat), jnp.float32),
            [pltpu.SemaphoreType.DMA] * 2,
        ],
    )
    def agg_kernel(tab0_hbm, tab1_hbm, src_hbm, dst_hbm, zrows_hbm, out_hbm,
                   src_v, dst_v, rows_v, stage_v, acc_sh, sem):
        cid = lax.axis_index("c")
        sid = lax.axis_index("s")
        wid = sid if feature_split else sid * NC + cid
        r0 = sid * rows_per_tile
        pc = 8
        for pi in range(rpw // pc):
            sl = pl.ds(pi * pc, pc)
            pltpu.async_copy(src_hbm.at[wid, sl], src_v.at[sl], sem[0])
            pltpu.async_copy(dst_hbm.at[wid, sl], dst_v.at[sl], sem[1])
        for pi in range(rpw // pc):
            sl = pl.ds(pi * pc, pc)
            pltpu.make_async_copy(src_hbm.at[wid, sl], src_v.at[sl],
                                  sem[0]).wait()
            pltpu.make_async_copy(dst_hbm.at[wid, sl], dst_v.at[sl],
                                  sem[1]).wait()
        pltpu.sync_copy(zrows_hbm, stage_v)
        for p in range(pieces):
            pltpu.sync_copy(stage_v, acc_sh.at[pl.ds(r0 + p * sub, sub)])
        plsc.subcore_barrier()

        def edge_loop(table_hbm):
            def fire(b, t):
                for j in range(KSUB):
                    pltpu.async_copy(table_hbm.at[src_v.at[t * KSUB + j]],
                                     rows_v[b].at[j], sem[b])

            def drain_scatter(b, t):
                for j in range(KSUB):
                    pltpu.make_async_copy(table_hbm.at[src_v.at[t * KSUB + j]],
                                          rows_v[b].at[j], sem[b]).wait()
                for j in range(KSUB):
                    pltpu.sync_copy(rows_v[b].at[j],
                                    acc_sh.at[dst_v.at[t * KSUB + j]],
                                    add=True)

            fire(0, 0)
            fire(1, 1)

            def body(i, carry):
                for b in range(2):
                    t = 2 * i + b
                    drain_scatter(b, t)
                    fire(b, t + 2)
                return carry

            lax.fori_loop(0, nsub // 2 - 1, body, 0)
            drain_scatter(0, nsub - 2)
            drain_scatter(1, nsub - 1)

        if feature_split:
            @pl.when(cid == 0)
            def _():
                edge_loop(tab0_hbm)

            @pl.when(cid == 1)
            def _():
                edge_loop(tab1_hbm)
        else:
            edge_loop(tab0_hbm)

        plsc.subcore_barrier()
        for p in range(pieces):
            sl = pl.ds(r0 + p * sub, sub)
            pltpu.sync_copy(acc_sh.at[sl], stage_v)
            pltpu.sync_copy(stage_v, out_hbm.at[cid, sl])

    return agg_kernel


def _make_agg_final_kernel(npad, rpad, feat):
    nworkers = NS
    rpw = rpad // nworkers
    nsub = rpw // KSUB
    rows_per_tile = npad // NS
    sub = _pick_sub(rows_per_tile, feat)
    pieces = rows_per_tile // sub
    mesh = plsc.VectorSubcoreMesh(core_axis_name="c", subcore_axis_name="s",
                                  num_cores=NC, num_subcores=NS)

    @functools.partial(
        pl.kernel, mesh=mesh,
        compiler_params=pltpu.CompilerParams(use_tc_tiling_on_sc=False),
        out_type=jax.ShapeDtypeStruct((npad, 2 * feat), jnp.float32),
        scratch_types=[
            pltpu.VMEM((rpw, LANE), jnp.int32),
            pltpu.VMEM((rpw, LANE), jnp.int32),
            [pltpu.VMEM((KSUB, LANE, feat), jnp.float32)] * 2,
            pltpu.VMEM((sub, feat), jnp.float32),
            pltpu.VMEM((sub, feat), jnp.float32),
            pltpu.VMEM((sub, 16), jnp.float32),
            pltpu.VMEM((2 * NC, 16), jnp.float32),
            pltpu.VMEM_SHARED((npad, feat), jnp.float32),
            [pltpu.SemaphoreType.DMA] * 2,
        ],
    )
    def agg_kernel(tab0_hbm, tab1_hbm, src_hbm, dst_hbm, zrows_hbm, dis_hbm,
                   bias_hbm, out_hbm, src_v, dst_v, rows_v, stage_v, hs_v,
                   dis_v, bias_v, acc_sh, sem):
        cid = lax.axis_index("c")
        sid = lax.axis_index("s")
        wid = sid
        r0 = sid * rows_per_tile
        pc = 8
        for pi in range(rpw // pc):
            sl = pl.ds(pi * pc, pc)
            pltpu.async_copy(src_hbm.at[wid, sl], src_v.at[sl], sem[0])
            pltpu.async_copy(dst_hbm.at[wid, sl], dst_v.at[sl], sem[1])
        for pi in range(rpw // pc):
            sl = pl.ds(pi * pc, pc)
            pltpu.make_async_copy(src_hbm.at[wid, sl], src_v.at[sl],
                                  sem[0]).wait()
            pltpu.make_async_copy(dst_hbm.at[wid, sl], dst_v.at[sl],
                                  sem[1]).wait()
        pltpu.sync_copy(zrows_hbm, stage_v)
        pltpu.sync_copy(bias_hbm, bias_v)
        for p in range(pieces):
            pltpu.sync_copy(stage_v, acc_sh.at[pl.ds(r0 + p * sub, sub)])
        plsc.subcore_barrier()

        def core_work(table_hbm, core):
            def fire(b, t):
                for j in range(KSUB):
                    pltpu.async_copy(table_hbm.at[src_v.at[t * KSUB + j]],
                                     rows_v[b].at[j], sem[b])

            def drain_scatter(b, t):
                for j in range(KSUB):
                    pltpu.make_async_copy(table_hbm.at[src_v.at[t * KSUB + j]],
                                          rows_v[b].at[j], sem[b]).wait()
                for j in range(KSUB):
                    pltpu.sync_copy(rows_v[b].at[j],
                                    acc_sh.at[dst_v.at[t * KSUB + j]],
                                    add=True)

            fire(0, 0)
            fire(1, 1)

            def body(i, carry):
                for b in range(2):
                    t = 2 * i + b
                    drain_scatter(b, t)
                    fire(b, t + 2)
                return carry

            lax.fori_loop(0, nsub // 2 - 1, body, 0)
            drain_scatter(0, nsub - 2)
            drain_scatter(1, nsub - 1)

            plsc.subcore_barrier()
            nh = feat // 16
            for p in range(pieces):
                sl = pl.ds(r0 + p * sub, sub)
                pltpu.sync_copy(acc_sh.at[sl], stage_v)
                pltpu.sync_copy(table_hbm.at[sl], hs_v)
                pltpu.sync_copy(dis_hbm.at[sl], dis_v)

                def rowbody(i, carry):
                    disv = dis_v[i]
                    for h in range(nh):
                        fs = pl.ds(h * 16, 16)
                        stage_v[i, fs] = ((stage_v[i, fs] + hs_v[i, fs])
                                          * disv + bias_v[core * nh + h])
                    return carry

                lax.fori_loop(0, sub, rowbody, 0)
                pltpu.sync_copy(stage_v,
                                out_hbm.at[sl, pl.ds(core * feat, feat)])

        @pl.when(cid == 0)
        def _():
            core_work(tab0_hbm, 0)

        @pl.when(cid == 1)
        def _():
            core_work(tab1_hbm, 1)

    return agg_kernel


def _tc1(x, w1, degp, bn):
    n, in_ch = x.shape
    hid = w1.shape[1]
    half = hid // 2

    def body(x_ref, w_ref, d_ref, lo_ref, hi_ref, dis_ref):
        deg = d_ref[0] + d_ref[1] + 1.0
        dis16 = lax.rsqrt(deg)
        dis_ref[...] = dis16
        h = jnp.dot(x_ref[...], w_ref[...], preferred_element_type=jnp.float32)
        hs = h * dis16[:, 0:1]
        lo_ref[...] = hs[:, :half]
        hi_ref[...] = hs[:, half:]

    return pl.pallas_call(
        body,
        grid=(n // bn,),
        in_specs=[
            pl.BlockSpec((bn, in_ch), lambda i: (i, 0)),
            pl.BlockSpec((in_ch, hid), lambda i: (0, 0)),
            pl.BlockSpec((NC, bn, 16), lambda i: (0, i, 0)),
        ],
        out_specs=[
            pl.BlockSpec((bn, half), lambda i: (i, 0)),
            pl.BlockSpec((bn, half), lambda i: (i, 0)),
            pl.BlockSpec((bn, 16), lambda i: (i, 0)),
        ],
        out_shape=[
            jax.ShapeDtypeStruct((n, half), jnp.float32),
            jax.ShapeDtypeStruct((n, half), jnp.float32),
            jax.ShapeDtypeStruct((n, 16), jnp.float32),
        ],
    )(x, w1, degp)


def _tc2(p, hs_lo, hs_hi, dis16, b1, w2, bn):
    n, half = hs_lo.shape
    dim = w2.shape[1]

    qh = dim // 2

    def body(p_ref, lo_ref, hi_ref, dis_ref, b_ref, w_ref, o_lo_ref,
             o_hi_ref):
        dis = dis_ref[:, 0:1]
        agg = jnp.concatenate(
            [p_ref[0] + lo_ref[...], p_ref[1] + hi_ref[...]], axis=1)
        z = jnp.maximum(agg * dis + b_ref[...], 0.0)
        hs2 = jnp.dot(z, w_ref[...], preferred_element_type=jnp.float32) * dis
        o_lo_ref[...] = hs2[:, :qh]
        o_hi_ref[...] = hs2[:, qh:]

    return pl.pallas_call(
        body,
        grid=(n // bn,),
        in_specs=[
            pl.BlockSpec((NC, bn, half), lambda i: (0, i, 0)),
            pl.BlockSpec((bn, half), lambda i: (i, 0)),
            pl.BlockSpec((bn, half), lambda i: (i, 0)),
            pl.BlockSpec((bn, 16), lambda i: (i, 0)),
            pl.BlockSpec((1, 2 * half), lambda i: (0, 0)),
            pl.BlockSpec((2 * half, dim), lambda i: (0, 0)),
        ],
        out_specs=[
            pl.BlockSpec((bn, qh), lambda i: (i, 0)),
            pl.BlockSpec((bn, qh), lambda i: (i, 0)),
        ],
        out_shape=[
            jax.ShapeDtypeStruct((n, qh), jnp.float32),
            jax.ShapeDtypeStruct((n, qh), jnp.float32),
        ],
    )(p, hs_lo, hs_hi, dis16, b1, w2)


def _tc3(q, hs2_lo, hs2_hi, dis16, b2, bn):
    n, qh = hs2_lo.shape
    dim = 2 * qh

    def body(q_ref, lo_ref, hi_ref, dis_ref, b_ref, o_ref):
        agg = jnp.concatenate(
            [q_ref[0] + lo_ref[...], q_ref[1] + hi_ref[...]], axis=1)
        o_ref[...] = agg * dis_ref[:, 0:1] + b_ref[...]

    return pl.pallas_call(
        body,
        grid=(n // bn,),
        in_specs=[
            pl.BlockSpec((NC, bn, qh), lambda i: (0, i, 0)),
            pl.BlockSpec((bn, qh), lambda i: (i, 0)),
            pl.BlockSpec((bn, qh), lambda i: (i, 0)),
            pl.BlockSpec((bn, 16), lambda i: (i, 0)),
            pl.BlockSpec((1, dim), lambda i: (0, 0)),
        ],
        out_specs=pl.BlockSpec((bn, dim), lambda i: (i, 0)),
        out_shape=jax.ShapeDtypeStruct((n, dim), jnp.float32),
    )(q, hs2_lo, hs2_hi, dis16, b2)


def kernel(x, edge_index, W1, b1, W2, b2):
    n, _ = x.shape
    hid = W1.shape[1]
    dim = W2.shape[1]
    e = edge_index.shape[1]
    assert e % LANE == 0
    r = e // LANE
    npad = _cdiv(n, NS * 8) * NS * 8
    rpad = _cdiv(r, 8 * NW) * 8 * NW
    bn = 1000 if n % 1000 == 0 else 8
    half = hid // 2

    src, dst = edge_index[0], edge_index[1]
    npadrows = rpad - r
    pad_e = npadrows * LANE
    pad_src = (jnp.arange(pad_e, dtype=jnp.int32) % n).reshape(npadrows, LANE)
    pad_dst = (n + jnp.arange(pad_e, dtype=jnp.int32) % (npad - n)).reshape(
        npadrows, LANE)
    srcp = jnp.concatenate([src.reshape(r, LANE), pad_src], axis=0)
    dstp = jnp.concatenate([dst.reshape(r, LANE), pad_dst], axis=0)
    ones16 = jnp.ones((LANE, 16), jnp.float32)

    rows_per_tile = npad // NS

    def zrows(feat):
        return jnp.zeros((_pick_sub(rows_per_tile, feat), feat), jnp.float32)

    srcp16 = srcp.reshape(NS, rpad // NS, LANE)
    dstp16 = dstp.reshape(NS, rpad // NS, LANE)
    srcp32 = srcp.reshape(NW, rpad // NW, LANE)
    dstp32 = dstp.reshape(NW, rpad // NW, LANE)
    degp = _make_deg_kernel(npad, rpad)(dstp32, ones16, zrows(16))
    hs_lo, hs_hi, dis16 = _tc1(x, W1, degp, bn)
    p = _make_agg_kernel(npad, rpad, half, feature_split=True)(
        hs_lo, hs_hi, srcp16, dstp16, zrows(half))
    hs2_lo, hs2_hi = _tc2(p, hs_lo, hs_hi, dis16, b1.reshape(1, hid), W2, bn)
    q = _make_agg_kernel(npad, rpad, dim // 2, feature_split=True)(
        hs2_lo, hs2_hi, srcp16, dstp16, zrows(dim // 2))
    return _tc3(q, hs2_lo, hs2_hi, dis16, b2.reshape(1, dim), bn)
